# baseline (speedup 1.0000x reference)
import functools

import jax
import jax.numpy as jnp
from jax import lax
from jax.experimental import pallas as pl
from jax.experimental.pallas import tpu as pltpu
from jax.experimental.pallas import tpu_sc as plsc

N = 10000
E = 320000
D = 128
H = 128
F_OUT = 10
B = 64

NC = 2
NS = 16
NW = NC * NS
EPW = E // NW
CH = 100
NCHUNK = EPW // CH
CPG = 20
G = NCHUNK // CPG
NBUF = 4
STRIPE = 640
STRIPE_LAST = N - 15 * STRIPE


def _agg_body(hp_hbm, srcr_hbm, dstr_hbm, zeros_hbm, out0_hbm, out1_hbm,
              src_v, dst_v, rows0, rows1, rows2, rows3, frows,
              sem0, sem1, sem2, sem3, agg_sh):
    rows_bufs = (rows0, rows1, rows2, rows3)
    sems = (sem0, sem1, sem2, sem3)
    c = lax.axis_index("c")
    s = lax.axis_index("s")
    wid = c * NS + s

    @pl.when(s < NS - 1)
    def _():
        pltpu.sync_copy(zeros_hbm, agg_sh.at[pl.ds(s * STRIPE, STRIPE)])

    @pl.when(s == NS - 1)
    def _():
        pltpu.sync_copy(zeros_hbm.at[pl.ds(0, STRIPE_LAST)],
                        agg_sh.at[pl.ds(15 * STRIPE, STRIPE_LAST)])

    plsc.subcore_barrier()

    def unpack_rows(rows):
        mask = jnp.full((16,), -65536, jnp.int32)

        def row5(r5, _):
            r0 = r5 * 5
            vs = [rows[r0 + dr, pl.ds(w * 16, 16)]
                  for dr in range(5) for w in range(4)]
            abs_ = [(plsc.bitcast(v << 16, jnp.float32),
                     plsc.bitcast(v & mask, jnp.float32)) for v in vs]
            for i, (a, b) in enumerate(abs_):
                dr, w = divmod(i, 4)
                frows[r0 + dr, pl.ds(w * 16, 16)] = a
                frows[r0 + dr, pl.ds(64 + w * 16, 16)] = b
            return 0

        lax.fori_loop(0, CH // 5, row5, 0)

    def group(g, _):
        pltpu.sync_copy(srcr_hbm.at[wid, pl.ds(g * CPG, CPG)], src_v)
        pltpu.sync_copy(dstr_hbm.at[wid, pl.ds(g * CPG, CPG)], dst_v)
        for b in range(NBUF):
            pltpu.async_copy(hp_hbm.at[src_v.at[b]], rows_bufs[b], sems[b])

        def chunk(j, _):
            for b in range(NBUF):
                @pl.when(j % NBUF == b)
                def _():
                    pltpu.make_async_copy(
                        hp_hbm.at[src_v.at[j]], rows_bufs[b], sems[b]).wait()
                    unpack_rows(rows_bufs[b])

                    @pl.when(j + NBUF < CPG)
                    def _():
                        pltpu.async_copy(hp_hbm.at[src_v.at[j + NBUF]],
                                         rows_bufs[b], sems[b])

                    pltpu.sync_copy(frows, agg_sh.at[dst_v.at[j]], add=True)

            return 0

        lax.fori_loop(0, CPG, chunk, 0)
        return 0

    lax.fori_loop(0, G, group, 0)
    plsc.subcore_barrier()

    stripe = pl.ds(s * STRIPE, STRIPE)
    last = pl.ds(15 * STRIPE, STRIPE_LAST)

    @pl.when(jnp.logical_and(c == 0, s < NS - 1))
    def _():
        pltpu.sync_copy(agg_sh.at[stripe], out0_hbm.at[stripe])

    @pl.when(jnp.logical_and(c == 0, s == NS - 1))
    def _():
        pltpu.sync_copy(agg_sh.at[last], out0_hbm.at[last])

    @pl.when(jnp.logical_and(c == 1, s < NS - 1))
    def _():
        pltpu.sync_copy(agg_sh.at[stripe], out1_hbm.at[stripe])

    @pl.when(jnp.logical_and(c == 1, s == NS - 1))
    def _():
        pltpu.sync_copy(agg_sh.at[last], out1_hbm.at[last])


@functools.cache
def _make_agg():
    mesh = plsc.VectorSubcoreMesh(core_axis_name="c", subcore_axis_name="s",
                                  num_cores=NC, num_subcores=NS)
    return pl.kernel(
        _agg_body,
        out_type=(jax.ShapeDtypeStruct((N, D), jnp.float32),
                  jax.ShapeDtypeStruct((N, D), jnp.float32)),
        mesh=mesh,
        compiler_params=pltpu.CompilerParams(needs_layout_passes=False,
                                             use_tc_tiling_on_sc=False),
        scratch_types=[
            pltpu.VMEM((CPG, CH), jnp.int32),
            pltpu.VMEM((CPG, CH), jnp.int32),
            pltpu.VMEM((CH, D // 2), jnp.int32),
            pltpu.VMEM((CH, D // 2), jnp.int32),
            pltpu.VMEM((CH, D // 2), jnp.int32),
            pltpu.VMEM((CH, D // 2), jnp.int32),
            pltpu.VMEM((CH, D), jnp.float32),
            pltpu.SemaphoreType.DMA,
            pltpu.SemaphoreType.DMA,
            pltpu.SemaphoreType.DMA,
            pltpu.SemaphoreType.DMA,
            pltpu.VMEM_SHARED((N, D), jnp.float32),
        ],
    )


def _agg(h, srcr, dstr, zeros):
    return _make_agg()(h, srcr, dstr, zeros)


TBLK = 1000
_HIGH = lax.Precision.HIGHEST


def _mlp_body(x_ref, p0_ref, p1_ref, wa_ref, ba_ref, wb_ref, bb_ref,
              o_ref, ob_ref):
    y = x_ref[...] + p0_ref[...] + p1_ref[...]
    h = jnp.dot(y, wa_ref[...], preferred_element_type=jnp.float32,
                precision=_HIGH) + ba_ref[...]
    h = jnp.maximum(h, 0.0)
    o = jnp.dot(h, wb_ref[...], preferred_element_type=jnp.float32,
                precision=_HIGH) + bb_ref[...]
    o = jnp.maximum(o, 0.0)
    o_ref[...] = o
    o16 = o.astype(jnp.bfloat16)
    lo = lax.bitcast_convert_type(o16[:, :D // 2], jnp.uint16) \
        .astype(jnp.uint32)
    hi = lax.bitcast_convert_type(o16[:, D // 2:], jnp.uint16) \
        .astype(jnp.uint32)
    ob_ref[...] = lax.bitcast_convert_type(lo | (hi << 16), jnp.int32)


def _mlp(x, p0, p1, wa, ba, wb, bb):
    grid = (N // TBLK,)
    return pl.pallas_call(
        _mlp_body,
        grid=grid,
        in_specs=[
            pl.BlockSpec((TBLK, D), lambda i: (i, 0)),
            pl.BlockSpec((TBLK, D), lambda i: (i, 0)),
            pl.BlockSpec((TBLK, D), lambda i: (i, 0)),
            pl.BlockSpec((D, H), lambda i: (0, 0)),
            pl.BlockSpec((1, H), lambda i: (0, 0)),
            pl.BlockSpec((H, H), lambda i: (0, 0)),
            pl.BlockSpec((1, H), lambda i: (0, 0)),
        ],
        out_specs=[pl.BlockSpec((TBLK, H), lambda i: (i, 0)),
                   pl.BlockSpec((TBLK, H // 2), lambda i: (i, 0))],
        out_shape=[jax.ShapeDtypeStruct((N, H), jnp.float32),
                   jax.ShapeDtypeStruct((N, H // 2), jnp.int32)],
    )(x, p0, p1, wa, ba, wb, bb)


def _head_body(oh_ref, h1_ref, h2_ref, h3_ref, wl1_ref, bl1_ref, wl2_ref,
               bl2_ref, o_ref, acc_ref, cnt_ref):
    i = pl.program_id(0)

    @pl.when(i == 0)
    def _():
        acc_ref[...] = jnp.zeros_like(acc_ref)
        cnt_ref[...] = jnp.zeros_like(cnt_ref)

    oh = oh_ref[...]
    dn = (((0,), (0,)), ((), ()))
    p1 = lax.dot_general(oh, h1_ref[...], dn,
                         preferred_element_type=jnp.float32, precision=_HIGH)
    p2 = lax.dot_general(oh, h2_ref[...], dn,
                         preferred_element_type=jnp.float32, precision=_HIGH)
    p3 = lax.dot_general(oh, h3_ref[...], dn,
                         preferred_element_type=jnp.float32, precision=_HIGH)
    acc_ref[...] += jnp.concatenate([p1, p2, p3], axis=1)
    cnt_ref[...] += jnp.sum(oh, axis=0, keepdims=True)

    @pl.when(i == pl.num_programs(0) - 1)
    def _():
        cnt = jnp.maximum(cnt_ref[...], 1.0)
        p = acc_ref[...] / cnt.reshape(B, 1)
        z = jnp.dot(p, wl1_ref[...], preferred_element_type=jnp.float32,
                    precision=_HIGH) + bl1_ref[...]
        z = jnp.maximum(z, 0.0)
        z = jnp.dot(z, wl2_ref[...], preferred_element_type=jnp.float32,
                    precision=_HIGH) + bl2_ref[...]
        m = jnp.max(z, axis=1, keepdims=True)
        lse = jnp.log(jnp.sum(jnp.exp(z - m), axis=1, keepdims=True)) + m
        o_ref[...] = z - lse


def _head(oh, h1, h2, h3, wl1, bl1, wl2, bl2):
    grid = (N // TBLK,)
    return pl.pallas_call(
        _head_body,
        grid=grid,
        in_specs=[
            pl.BlockSpec((TBLK, B), lambda i: (i, 0)),
            pl.BlockSpec((TBLK, H), lambda i: (i, 0)),
            pl.BlockSpec((TBLK, H), lambda i: (i, 0)),
            pl.BlockSpec((TBLK, H), lambda i: (i, 0)),
            pl.BlockSpec((3 * H, 3 * H), lambda i: (0, 0)),
            pl.BlockSpec((1, 3 * H), lambda i: (0, 0)),
            pl.BlockSpec((3 * H, F_OUT), lambda i: (0, 0)),
            pl.BlockSpec((1, F_OUT), lambda i: (0, 0)),
        ],
        out_specs=pl.BlockSpec((B, F_OUT), lambda i: (0, 0)),
        out_shape=jax.ShapeDtypeStruct((B, F_OUT), jnp.float32),
        scratch_shapes=[
            pltpu.VMEM((B, 3 * H), jnp.float32),
            pltpu.VMEM((1, B), jnp.float32),
        ],
    )(oh, h1, h2, h3, wl1, bl1, wl2, bl2)


def _fold_bn(wa, ba, g, bt, rm, rv):
    s = g / jnp.sqrt(rv + 1e-5)
    return wa * s[None, :], ((ba - rm) * s + bt)[None, :]


def kernel(x, edge_index, batch,
           Wa1, ba1, g1, bt1, rm1, rv1, Wb1, bb1,
           Wa2, ba2, g2, bt2, rm2, rv2, Wb2, bb2,
           Wa3, ba3, g3, bt3, rm3, rv3, Wb3, bb3,
           Wl1, bl1, Wl2, bl2):
    srcr = edge_index[0].reshape(NW, NCHUNK, CH)
    dstr = edge_index[1].reshape(NW, NCHUNK, CH)
    zeros = jnp.zeros((STRIPE, D), jnp.float32)
    oh = (batch[:, None] == jnp.arange(B, dtype=batch.dtype)[None, :]) \
        .astype(jnp.float32)

    wa1, ba1f = _fold_bn(Wa1, ba1, g1, bt1, rm1, rv1)
    wa2, ba2f = _fold_bn(Wa2, ba2, g2, bt2, rm2, rv2)
    wa3, ba3f = _fold_bn(Wa3, ba3, g3, bt3, rm3, rv3)

    x16 = x.astype(jnp.bfloat16)
    xlo = lax.bitcast_convert_type(x16[:, :D // 2], jnp.uint16) \
        .astype(jnp.uint32)
    xhi = lax.bitcast_convert_type(x16[:, D // 2:], jnp.uint16) \
        .astype(jnp.uint32)
    xp = lax.bitcast_convert_type(xlo | (xhi << 16), jnp.int32)

    a0, a1 = _agg(xp, srcr, dstr, zeros)
    h1, h1p = _mlp(x, a0, a1, wa1, ba1f, Wb1, bb1[None, :])
    b0, b1 = _agg(h1p, srcr, dstr, zeros)
    h2, h2p = _mlp(h1, b0, b1, wa2, ba2f, Wb2, bb2[None, :])
    c0, c1 = _agg(h2p, srcr, dstr, zeros)
    h3, _ = _mlp(h2, c0, c1, wa3, ba3f, Wb3, bb3[None, :])

    return _head(oh, h1, h2, h3, Wl1, bl1[None, :], Wl2, bl2[None, :])


# speedup vs baseline: 1.3244x; 1.3244x over previous
import functools

import jax
import jax.numpy as jnp
from jax import lax
from jax.experimental import pallas as pl
from jax.experimental.pallas import tpu as pltpu
from jax.experimental.pallas import tpu_sc as plsc

N = 10000
E = 320000
D = 128
H = 128
F_OUT = 10
B = 64

NC = 2
NS = 16
NW = NC * NS
EPW = E // NW
CH = 100
NCHUNK = EPW // CH
CPG = 20
G = NCHUNK // CPG
NBUF = 3
STRIPE = 640
STRIPE_LAST = N - 15 * STRIPE


def _agg_body(h_hbm, srcr_hbm, dstr_hbm, zeros_hbm, out0_hbm, out1_hbm,
              src_v, dst_v, rows0, rows1, rows2,
              semg0, semg1, semg2, sems0, sems1, sems2, agg_sh):
    rows_bufs = (rows0, rows1, rows2)
    semg = (semg0, semg1, semg2)
    sems = (sems0, sems1, sems2)
    c = lax.axis_index("c")
    s = lax.axis_index("s")
    wid = c * NS + s

    @pl.when(s < NS - 1)
    def _():
        pltpu.sync_copy(zeros_hbm, agg_sh.at[pl.ds(s * STRIPE, STRIPE)])

    @pl.when(s == NS - 1)
    def _():
        pltpu.sync_copy(zeros_hbm.at[pl.ds(0, STRIPE_LAST)],
                        agg_sh.at[pl.ds(15 * STRIPE, STRIPE_LAST)])

    plsc.subcore_barrier()

    def group(g, _):
        pltpu.sync_copy(srcr_hbm.at[wid, pl.ds(g * CPG, CPG)], src_v)
        pltpu.sync_copy(dstr_hbm.at[wid, pl.ds(g * CPG, CPG)], dst_v)
        for b in range(2):
            pltpu.async_copy(h_hbm.at[src_v.at[b]], rows_bufs[b], semg[b])

        def chunk(j, _):
            for b in range(NBUF):
                @pl.when(j % NBUF == b)
                def _():
                    pltpu.make_async_copy(
                        h_hbm.at[src_v.at[j]], rows_bufs[b], semg[b]).wait()
                    pltpu.async_copy(rows_bufs[b],
                                     agg_sh.at[dst_v.at[j]], sems[b],
                                     add=True)

                    b2 = (b + 2) % NBUF

                    @pl.when(j + 2 < CPG)
                    def _():
                        @pl.when(j >= 1)
                        def _():
                            pltpu.make_async_copy(
                                rows_bufs[b2],
                                agg_sh.at[dst_v.at[j - 1]], sems[b2]).wait()

                        pltpu.async_copy(h_hbm.at[src_v.at[j + 2]],
                                         rows_bufs[b2], semg[b2])

            return 0

        lax.fori_loop(0, CPG, chunk, 0)
        for b in range(NBUF):
            lastj = CPG - 1 - ((CPG - 1 - b) % NBUF)
            pltpu.make_async_copy(
                rows_bufs[b], agg_sh.at[dst_v.at[lastj]], sems[b]).wait()
        return 0

    lax.fori_loop(0, G, group, 0)
    plsc.subcore_barrier()

    stripe = pl.ds(s * STRIPE, STRIPE)
    last = pl.ds(15 * STRIPE, STRIPE_LAST)

    @pl.when(jnp.logical_and(c == 0, s < NS - 1))
    def _():
        pltpu.sync_copy(agg_sh.at[stripe], out0_hbm.at[stripe])

    @pl.when(jnp.logical_and(c == 0, s == NS - 1))
    def _():
        pltpu.sync_copy(agg_sh.at[last], out0_hbm.at[last])

    @pl.when(jnp.logical_and(c == 1, s < NS - 1))
    def _():
        pltpu.sync_copy(agg_sh.at[stripe], out1_hbm.at[stripe])

    @pl.when(jnp.logical_and(c == 1, s == NS - 1))
    def _():
        pltpu.sync_copy(agg_sh.at[last], out1_hbm.at[last])


@functools.cache
def _make_agg():
    mesh = plsc.VectorSubcoreMesh(core_axis_name="c", subcore_axis_name="s",
                                  num_cores=NC, num_subcores=NS)
    return pl.kernel(
        _agg_body,
        out_type=(jax.ShapeDtypeStruct((N, D), jnp.float32),
                  jax.ShapeDtypeStruct((N, D), jnp.float32)),
        mesh=mesh,
        compiler_params=pltpu.CompilerParams(needs_layout_passes=False,
                                             use_tc_tiling_on_sc=False),
        scratch_types=[
            pltpu.VMEM((CPG, CH), jnp.int32),
            pltpu.VMEM((CPG, CH), jnp.int32),
            pltpu.VMEM((CH, D), jnp.float32),
            pltpu.VMEM((CH, D), jnp.float32),
            pltpu.VMEM((CH, D), jnp.float32),
            pltpu.SemaphoreType.DMA,
            pltpu.SemaphoreType.DMA,
            pltpu.SemaphoreType.DMA,
            pltpu.SemaphoreType.DMA,
            pltpu.SemaphoreType.DMA,
            pltpu.SemaphoreType.DMA,
            pltpu.VMEM_SHARED((N, D), jnp.float32),
        ],
    )


def _agg(h, srcr, dstr, zeros):
    return _make_agg()(h, srcr, dstr, zeros)


TBLK = 1000
_HIGH = lax.Precision.HIGHEST


def _mlp_body(x_ref, p0_ref, p1_ref, wa_ref, ba_ref, wb_ref, bb_ref, o_ref):
    y = x_ref[...] + p0_ref[...] + p1_ref[...]
    h = jnp.dot(y, wa_ref[...], preferred_element_type=jnp.float32,
                precision=_HIGH) + ba_ref[...]
    h = jnp.maximum(h, 0.0)
    o = jnp.dot(h, wb_ref[...], preferred_element_type=jnp.float32,
                precision=_HIGH) + bb_ref[...]
    o_ref[...] = jnp.maximum(o, 0.0)


def _mlp(x, p0, p1, wa, ba, wb, bb):
    grid = (N // TBLK,)
    return pl.pallas_call(
        _mlp_body,
        grid=grid,
        in_specs=[
            pl.BlockSpec((TBLK, D), lambda i: (i, 0)),
            pl.BlockSpec((TBLK, D), lambda i: (i, 0)),
            pl.BlockSpec((TBLK, D), lambda i: (i, 0)),
            pl.BlockSpec((D, H), lambda i: (0, 0)),
            pl.BlockSpec((1, H), lambda i: (0, 0)),
            pl.BlockSpec((H, H), lambda i: (0, 0)),
            pl.BlockSpec((1, H), lambda i: (0, 0)),
        ],
        out_specs=pl.BlockSpec((TBLK, H), lambda i: (i, 0)),
        out_shape=jax.ShapeDtypeStruct((N, H), jnp.float32),
    )(x, p0, p1, wa, ba, wb, bb)


def _head_body(oh_ref, h1_ref, h2_ref, h3_ref, wl1_ref, bl1_ref, wl2_ref,
               bl2_ref, o_ref, acc_ref, cnt_ref):
    i = pl.program_id(0)

    @pl.when(i == 0)
    def _():
        acc_ref[...] = jnp.zeros_like(acc_ref)
        cnt_ref[...] = jnp.zeros_like(cnt_ref)

    oh = oh_ref[...]
    dn = (((0,), (0,)), ((), ()))
    p1 = lax.dot_general(oh, h1_ref[...], dn,
                         preferred_element_type=jnp.float32, precision=_HIGH)
    p2 = lax.dot_general(oh, h2_ref[...], dn,
                         preferred_element_type=jnp.float32, precision=_HIGH)
    p3 = lax.dot_general(oh, h3_ref[...], dn,
                         preferred_element_type=jnp.float32, precision=_HIGH)
    acc_ref[...] += jnp.concatenate([p1, p2, p3], axis=1)
    cnt_ref[...] += jnp.sum(oh, axis=0, keepdims=True)

    @pl.when(i == pl.num_programs(0) - 1)
    def _():
        cnt = jnp.maximum(cnt_ref[...], 1.0)
        p = acc_ref[...] / cnt.reshape(B, 1)
        z = jnp.dot(p, wl1_ref[...], preferred_element_type=jnp.float32,
                    precision=_HIGH) + bl1_ref[...]
        z = jnp.maximum(z, 0.0)
        z = jnp.dot(z, wl2_ref[...], preferred_element_type=jnp.float32,
                    precision=_HIGH) + bl2_ref[...]
        m = jnp.max(z, axis=1, keepdims=True)
        lse = jnp.log(jnp.sum(jnp.exp(z - m), axis=1, keepdims=True)) + m
        o_ref[...] = z - lse


def _head(oh, h1, h2, h3, wl1, bl1, wl2, bl2):
    grid = (N // TBLK,)
    return pl.pallas_call(
        _head_body,
        grid=grid,
        in_specs=[
            pl.BlockSpec((TBLK, B), lambda i: (i, 0)),
            pl.BlockSpec((TBLK, H), lambda i: (i, 0)),
            pl.BlockSpec((TBLK, H), lambda i: (i, 0)),
            pl.BlockSpec((TBLK, H), lambda i: (i, 0)),
            pl.BlockSpec((3 * H, 3 * H), lambda i: (0, 0)),
            pl.BlockSpec((1, 3 * H), lambda i: (0, 0)),
            pl.BlockSpec((3 * H, F_OUT), lambda i: (0, 0)),
            pl.BlockSpec((1, F_OUT), lambda i: (0, 0)),
        ],
        out_specs=pl.BlockSpec((B, F_OUT), lambda i: (0, 0)),
        out_shape=jax.ShapeDtypeStruct((B, F_OUT), jnp.float32),
        scratch_shapes=[
            pltpu.VMEM((B, 3 * H), jnp.float32),
            pltpu.VMEM((1, B), jnp.float32),
        ],
    )(oh, h1, h2, h3, wl1, bl1, wl2, bl2)


def _fold_bn(wa, ba, g, bt, rm, rv):
    s = g / jnp.sqrt(rv + 1e-5)
    return wa * s[None, :], ((ba - rm) * s + bt)[None, :]


def kernel(x, edge_index, batch,
           Wa1, ba1, g1, bt1, rm1, rv1, Wb1, bb1,
           Wa2, ba2, g2, bt2, rm2, rv2, Wb2, bb2,
           Wa3, ba3, g3, bt3, rm3, rv3, Wb3, bb3,
           Wl1, bl1, Wl2, bl2):
    srcr = edge_index[0].reshape(NW, NCHUNK, CH)
    dstr = edge_index[1].reshape(NW, NCHUNK, CH)
    zeros = jnp.zeros((STRIPE, D), jnp.float32)
    oh = (batch[:, None] == jnp.arange(B, dtype=batch.dtype)[None, :]) \
        .astype(jnp.float32)

    wa1, ba1f = _fold_bn(Wa1, ba1, g1, bt1, rm1, rv1)
    wa2, ba2f = _fold_bn(Wa2, ba2, g2, bt2, rm2, rv2)
    wa3, ba3f = _fold_bn(Wa3, ba3, g3, bt3, rm3, rv3)

    a0, a1 = _agg(x, srcr, dstr, zeros)
    h1 = _mlp(x, a0, a1, wa1, ba1f, Wb1, bb1[None, :])
    b0, b1 = _agg(h1, srcr, dstr, zeros)
    h2 = _mlp(h1, b0, b1, wa2, ba2f, Wb2, bb2[None, :])
    c0, c1 = _agg(h2, srcr, dstr, zeros)
    h3 = _mlp(h2, c0, c1, wa3, ba3f, Wb3, bb3[None, :])

    return _head(oh, h1, h2, h3, Wl1, bl1[None, :], Wl2, bl2[None, :])


# speedup vs baseline: 1.3927x; 1.0516x over previous
import functools

import jax
import jax.numpy as jnp
from jax import lax
from jax.experimental import pallas as pl
from jax.experimental.pallas import tpu as pltpu
from jax.experimental.pallas import tpu_sc as plsc

N = 10000
E = 320000
D = 128
H = 128
F_OUT = 10
B = 64

NC = 2
NS = 16
NW = NC * NS
EPW = E // NW
CH = 125
NCHUNK = EPW // CH
CPG = 8
G = NCHUNK // CPG
NBUF = 3
STRIPE = 640
STRIPE_LAST = N - 15 * STRIPE


def _agg_body(h_hbm, srcr_hbm, dstr_hbm, zeros_hbm, out0_hbm, out1_hbm,
              src_v, dst_v, rows0, rows1, rows2,
              semg0, semg1, semg2, sems0, sems1, sems2, agg_sh):
    rows_bufs = (rows0, rows1, rows2)
    semg = (semg0, semg1, semg2)
    sems = (sems0, sems1, sems2)
    c = lax.axis_index("c")
    s = lax.axis_index("s")
    wid = c * NS + s

    @pl.when(s < NS - 1)
    def _():
        pltpu.sync_copy(zeros_hbm, agg_sh.at[pl.ds(s * STRIPE, STRIPE)])

    @pl.when(s == NS - 1)
    def _():
        pltpu.sync_copy(zeros_hbm.at[pl.ds(0, STRIPE_LAST)],
                        agg_sh.at[pl.ds(15 * STRIPE, STRIPE_LAST)])

    plsc.subcore_barrier()

    def group(g, _):
        pltpu.sync_copy(srcr_hbm.at[wid, pl.ds(g * CPG, CPG)], src_v)
        pltpu.sync_copy(dstr_hbm.at[wid, pl.ds(g * CPG, CPG)], dst_v)
        for b in range(2):
            pltpu.async_copy(h_hbm.at[src_v.at[b]], rows_bufs[b], semg[b])

        def chunk(j, _):
            for b in range(NBUF):
                @pl.when(j % NBUF == b)
                def _():
                    pltpu.make_async_copy(
                        h_hbm.at[src_v.at[j]], rows_bufs[b], semg[b]).wait()
                    pltpu.async_copy(rows_bufs[b],
                                     agg_sh.at[dst_v.at[j]], sems[b],
                                     add=True)

                    b2 = (b + 2) % NBUF

                    @pl.when(j + 2 < CPG)
                    def _():
                        @pl.when(j >= 1)
                        def _():
                            pltpu.make_async_copy(
                                rows_bufs[b2],
                                agg_sh.at[dst_v.at[j - 1]], sems[b2]).wait()

                        pltpu.async_copy(h_hbm.at[src_v.at[j + 2]],
                                         rows_bufs[b2], semg[b2])

            return 0

        lax.fori_loop(0, CPG, chunk, 0)
        for b in range(NBUF):
            lastj = CPG - 1 - ((CPG - 1 - b) % NBUF)
            pltpu.make_async_copy(
                rows_bufs[b], agg_sh.at[dst_v.at[lastj]], sems[b]).wait()
        return 0

    lax.fori_loop(0, G, group, 0)
    plsc.subcore_barrier()

    stripe = pl.ds(s * STRIPE, STRIPE)
    last = pl.ds(15 * STRIPE, STRIPE_LAST)

    @pl.when(jnp.logical_and(c == 0, s < NS - 1))
    def _():
        pltpu.sync_copy(agg_sh.at[stripe], out0_hbm.at[stripe])

    @pl.when(jnp.logical_and(c == 0, s == NS - 1))
    def _():
        pltpu.sync_copy(agg_sh.at[last], out0_hbm.at[last])

    @pl.when(jnp.logical_and(c == 1, s < NS - 1))
    def _():
        pltpu.sync_copy(agg_sh.at[stripe], out1_hbm.at[stripe])

    @pl.when(jnp.logical_and(c == 1, s == NS - 1))
    def _():
        pltpu.sync_copy(agg_sh.at[last], out1_hbm.at[last])


@functools.cache
def _make_agg():
    mesh = plsc.VectorSubcoreMesh(core_axis_name="c", subcore_axis_name="s",
                                  num_cores=NC, num_subcores=NS)
    return pl.kernel(
        _agg_body,
        out_type=(jax.ShapeDtypeStruct((N, D), jnp.float32),
                  jax.ShapeDtypeStruct((N, D), jnp.float32)),
        mesh=mesh,
        compiler_params=pltpu.CompilerParams(needs_layout_passes=False,
                                             use_tc_tiling_on_sc=False),
        scratch_types=[
            pltpu.VMEM((CPG, CH), jnp.int32),
            pltpu.VMEM((CPG, CH), jnp.int32),
            pltpu.VMEM((CH, D), jnp.float32),
            pltpu.VMEM((CH, D), jnp.float32),
            pltpu.VMEM((CH, D), jnp.float32),
            pltpu.SemaphoreType.DMA,
            pltpu.SemaphoreType.DMA,
            pltpu.SemaphoreType.DMA,
            pltpu.SemaphoreType.DMA,
            pltpu.SemaphoreType.DMA,
            pltpu.SemaphoreType.DMA,
            pltpu.VMEM_SHARED((N, D), jnp.float32),
        ],
    )


def _agg(h, srcr, dstr, zeros):
    return _make_agg()(h, srcr, dstr, zeros)


TBLK = 1000
_HIGH = lax.Precision.DEFAULT


def _mlp_body(x_ref, p0_ref, p1_ref, oh_ref, wa_ref, ba_ref, wb_ref, bb_ref,
              o_ref, pool_ref, cnt_ref, accp_ref, accc_ref):
    i = pl.program_id(0)

    @pl.when(i == 0)
    def _():
        accp_ref[...] = jnp.zeros_like(accp_ref)
        accc_ref[...] = jnp.zeros_like(accc_ref)

    y = x_ref[...] + p0_ref[...] + p1_ref[...]
    h = jnp.dot(y, wa_ref[...], preferred_element_type=jnp.float32,
                precision=_HIGH) + ba_ref[...]
    h = jnp.maximum(h, 0.0)
    o = jnp.dot(h, wb_ref[...], preferred_element_type=jnp.float32,
                precision=_HIGH) + bb_ref[...]
    o = jnp.maximum(o, 0.0)
    o_ref[...] = o

    oh = oh_ref[...]
    dn = (((0,), (0,)), ((), ()))
    accp_ref[...] += lax.dot_general(oh, o, dn,
                                     preferred_element_type=jnp.float32,
                                     precision=_HIGH)
    accc_ref[...] += jnp.sum(oh, axis=0, keepdims=True)

    @pl.when(i == pl.num_programs(0) - 1)
    def _():
        pool_ref[...] = accp_ref[...]
        cnt_ref[...] = accc_ref[...]


def _mlp(x, p0, p1, oh, wa, ba, wb, bb):
    grid = (N // TBLK,)
    return pl.pallas_call(
        _mlp_body,
        grid=grid,
        in_specs=[
            pl.BlockSpec((TBLK, D), lambda i: (i, 0)),
            pl.BlockSpec((TBLK, D), lambda i: (i, 0)),
            pl.BlockSpec((TBLK, D), lambda i: (i, 0)),
            pl.BlockSpec((TBLK, B), lambda i: (i, 0)),
            pl.BlockSpec((D, H), lambda i: (0, 0)),
            pl.BlockSpec((1, H), lambda i: (0, 0)),
            pl.BlockSpec((H, H), lambda i: (0, 0)),
            pl.BlockSpec((1, H), lambda i: (0, 0)),
        ],
        out_specs=[pl.BlockSpec((TBLK, H), lambda i: (i, 0)),
                   pl.BlockSpec((B, H), lambda i: (0, 0)),
                   pl.BlockSpec((1, B), lambda i: (0, 0))],
        out_shape=[jax.ShapeDtypeStruct((N, H), jnp.float32),
                   jax.ShapeDtypeStruct((B, H), jnp.float32),
                   jax.ShapeDtypeStruct((1, B), jnp.float32)],
        scratch_shapes=[
            pltpu.VMEM((B, H), jnp.float32),
            pltpu.VMEM((1, B), jnp.float32),
        ],
    )(x, p0, p1, oh, wa, ba, wb, bb)


def _head_body(p1_ref, p2_ref, p3_ref, cnt_ref, wl1_ref, bl1_ref, wl2_ref,
               bl2_ref, o_ref):
    cnt = jnp.maximum(cnt_ref[...], 1.0).reshape(B, 1)
    p = jnp.concatenate([p1_ref[...], p2_ref[...], p3_ref[...]], axis=1) / cnt
    z = jnp.dot(p, wl1_ref[...], preferred_element_type=jnp.float32,
                precision=_HIGH) + bl1_ref[...]
    z = jnp.maximum(z, 0.0)
    z = jnp.dot(z, wl2_ref[...], preferred_element_type=jnp.float32,
                precision=_HIGH) + bl2_ref[...]
    m = jnp.max(z, axis=1, keepdims=True)
    lse = jnp.log(jnp.sum(jnp.exp(z - m), axis=1, keepdims=True)) + m
    o_ref[...] = z - lse


def _head(p1, p2, p3, cnt, wl1, bl1, wl2, bl2):
    return pl.pallas_call(
        _head_body,
        out_shape=jax.ShapeDtypeStruct((B, F_OUT), jnp.float32),
    )(p1, p2, p3, cnt, wl1, bl1, wl2, bl2)


def _fold_bn(wa, ba, g, bt, rm, rv):
    s = g / jnp.sqrt(rv + 1e-5)
    return wa * s[None, :], ((ba - rm) * s + bt)[None, :]


def kernel(x, edge_index, batch,
           Wa1, ba1, g1, bt1, rm1, rv1, Wb1, bb1,
           Wa2, ba2, g2, bt2, rm2, rv2, Wb2, bb2,
           Wa3, ba3, g3, bt3, rm3, rv3, Wb3, bb3,
           Wl1, bl1, Wl2, bl2):
    srcr = edge_index[0].reshape(NW, NCHUNK, CH)
    dstr = edge_index[1].reshape(NW, NCHUNK, CH)
    zeros = jnp.zeros((STRIPE, D), jnp.float32)
    oh = (batch[:, None] == jnp.arange(B, dtype=batch.dtype)[None, :]) \
        .astype(jnp.float32)

    wa1, ba1f = _fold_bn(Wa1, ba1, g1, bt1, rm1, rv1)
    wa2, ba2f = _fold_bn(Wa2, ba2, g2, bt2, rm2, rv2)
    wa3, ba3f = _fold_bn(Wa3, ba3, g3, bt3, rm3, rv3)

    a0, a1 = _agg(x, srcr, dstr, zeros)
    h1, p1, cnt = _mlp(x, a0, a1, oh, wa1, ba1f, Wb1, bb1[None, :])
    b0, b1 = _agg(h1, srcr, dstr, zeros)
    h2, p2, _ = _mlp(h1, b0, b1, oh, wa2, ba2f, Wb2, bb2[None, :])
    c0, c1 = _agg(h2, srcr, dstr, zeros)
    h3, p3, _ = _mlp(h2, c0, c1, oh, wa3, ba3f, Wb3, bb3[None, :])

    return _head(p1, p2, p3, cnt, Wl1, bl1[None, :], Wl2, bl2[None, :])
